# baseline (speedup 1.0000x reference)
import functools
import math

import jax
import jax.numpy as jnp
from jax import lax
from jax.experimental import pallas as pl
from jax.experimental.pallas import tpu as pltpu
from jax.experimental.pallas import tpu_sc as plsc

VOCAB = 1000000
EMB = 64
PAD = 128
SCALE = math.sqrt(EMB)

NUM_WORKERS = 32
BDIM = 4096
TDIM = 200
B_TOTAL = BDIM * TDIM
PER_W = B_TOTAL // NUM_WORKERS
CHUNK = 80
SEG = 40
NCHUNK = PER_W // CHUNK
NBUF = 2
GROUP = 2 * NBUF
SETTOK = NBUF * CHUNK
NBODY = NCHUNK // GROUP
LANES = 16


def _make_kernel():
  mesh = plsc.VectorSubcoreMesh(core_axis_name="c", subcore_axis_name="s")

  rows_scratch = [pltpu.VMEM((CHUNK, PAD), jnp.float32)
                  for _ in range(2 * NBUF)]
  obuf_scratch = [pltpu.VMEM((CHUNK, EMB), jnp.float32)
                  for _ in range(2 * NBUF)]
  idx_scratch = [pltpu.VMEM((SETTOK,), jnp.int32) for _ in range(4)]
  gsem_scratch = [pltpu.SemaphoreType.DMA for _ in range(2 * NBUF)]

  @functools.partial(
      pl.kernel,
      mesh=mesh,
      out_type=jax.ShapeDtypeStruct((BDIM, TDIM, EMB), jnp.float32),
      scratch_types=rows_scratch + obuf_scratch + idx_scratch + gsem_scratch
      + [pltpu.SemaphoreType.DMA, pltpu.SemaphoreType.DMA],
  )
  def emb_kernel(tokens_hbm, table_hbm, out_hbm, *scratch):
    rows = scratch[:2 * NBUF]
    obuf = scratch[2 * NBUF:4 * NBUF]
    idxraw = scratch[4 * NBUF:4 * NBUF + 2]
    idx2 = scratch[4 * NBUF + 2:4 * NBUF + 4]
    gsem = scratch[4 * NBUF + 4:6 * NBUF + 4]
    osem = scratch[6 * NBUF + 4:]
    rows_ab = (rows[:NBUF], rows[NBUF:])
    obuf_ab = (obuf[:NBUF], obuf[NBUF:])
    gsem_ab = (gsem[:NBUF], gsem[NBUF:])

    wid = lax.axis_index("s") * 2 + lax.axis_index("c")
    base = wid * PER_W

    def fetch_idx(s, g):
      off = base + g * GROUP * CHUNK + s * SETTOK
      pltpu.sync_copy(tokens_hbm.at[pl.ds(off, SETTOK)], idxraw[s])
      def shift_body(v, carry):
        sl = pl.ds(v * LANES, LANES)
        idx2[s][sl] = lax.shift_right_logical(idxraw[s][sl], 1)
        return carry
      lax.fori_loop(0, SETTOK // LANES, shift_body, 0, unroll=2)

    for s in range(2):
      fetch_idx(s, 0)

    def repack(src, dst, s, loff):
      def group_body(jg, carry):
        j0 = jg * LANES
        par_vec = lax.rem(idxraw[s][pl.ds(loff + j0, LANES)], 2) * EMB
        for jj in range(LANES):
          start = par_vec[jj]
          for i in range(EMB // LANES):
            dst[j0 + jj, pl.ds(i * LANES, LANES)] = (
                src[j0 + jj, pl.ds(start + i * LANES, LANES)] * SCALE)
        return carry
      lax.fori_loop(0, CHUNK // LANES, group_body, 0)

    def body(g, carry):
      goff = g * GROUP * CHUNK
      handles = [None] * 2
      for s in range(2):
        @pl.when(g > 0)
        def _(s=s):
          for b in range(NBUF):
            for seg in range(CHUNK // SEG):
              pltpu.make_async_copy(
                  obuf_ab[s][b].at[pl.ds(seg * SEG, SEG)],
                  out_hbm.at[0, pl.ds(0, SEG)], osem[s]).wait()
        handles[s] = [
            pltpu.async_copy(
                table_hbm.at[idx2[s].at[pl.ds(b * CHUNK, CHUNK)]],
                rows_ab[s][b], gsem_ab[s][b])
            for b in range(NBUF)
        ]
      for s in range(2):
        for b in range(NBUF):
          handles[s][b].wait()
          repack(rows_ab[s][b], obuf_ab[s][b], s, b * CHUNK)
          for seg in range(CHUNK // SEG):
            row0 = base + goff + (s * NBUF + b) * CHUNK + seg * SEG
            brow = row0 // TDIM
            t0 = pl.multiple_of(row0 - brow * TDIM, 8)
            pltpu.async_copy(
                obuf_ab[s][b].at[pl.ds(seg * SEG, SEG)],
                out_hbm.at[brow, pl.ds(t0, SEG)], osem[s])
        @pl.when(g < NBODY - 1)
        def _(s=s, g=g):
          fetch_idx(s, g + 1)
      return carry

    lax.fori_loop(0, NBODY, body, 0)
    for s in range(2):
      for b in range(NBUF):
        for seg in range(CHUNK // SEG):
          pltpu.make_async_copy(
              obuf_ab[s][b].at[pl.ds(seg * SEG, SEG)],
              out_hbm.at[0, pl.ds(0, SEG)], osem[s]).wait()

  return emb_kernel


_emb_kernel = _make_kernel()


def kernel(tokens, table):
  flat = tokens.reshape(-1).astype(jnp.int32)
  tbl2 = table.reshape(VOCAB // 2, PAD)
  return _emb_kernel(flat, tbl2)
